# Initial kernel scaffold; baseline (speedup 1.0000x reference)
import jax
import jax.numpy as jnp
from jax.experimental import pallas as pl


def kernel(x, pos, mi_W, mi_b, tb0_lin_in_W, tb0_lin_in_b, tb0_lin_out_W, tb0_lin_out_b, tb0_pos_W1, tb0_pos_b1, tb0_pos_W2, tb0_pos_b2, tb0_attn_W1, tb0_attn_b1, tb0_attn_W2, tb0_attn_b2, tb0_lin_W, tb0_src_W, tb0_dst_W, tb1_lin_in_W, tb1_lin_in_b, tb1_lin_out_W, tb1_lin_out_b, tb1_pos_W1, tb1_pos_b1, tb1_pos_W2, tb1_pos_b2, tb1_attn_W1, tb1_attn_b1, tb1_attn_W2, tb1_attn_b2, tb1_lin_W, tb1_src_W, tb1_dst_W, tb2_lin_in_W, tb2_lin_in_b, tb2_lin_out_W, tb2_lin_out_b, tb2_pos_W1, tb2_pos_b1, tb2_pos_W2, tb2_pos_b2, tb2_attn_W1, tb2_attn_b1, tb2_attn_W2, tb2_attn_b2, tb2_lin_W, tb2_src_W, tb2_dst_W, td0_W, td0_b, td1_W, td1_b, mo_W1, mo_b1, mo_W2, mo_b2):
    raise NotImplementedError("write your pallas kernel here")


# speedup vs baseline: 7.8014x; 7.8014x over previous
import functools

import jax
import jax.numpy as jnp
from jax import lax
from jax.experimental import pallas as pl
from jax.experimental.pallas import tpu as pltpu
from jax.experimental.pallas import tpu_sc as plsc

R = 128
K = 16
E = K + 1
_BIG = 2 ** 30


def _pad_rows(a, n_pad):
    n = a.shape[0]
    if n == n_pad:
        return a
    return jnp.pad(a, ((0, n_pad - n),) + ((0, 0),) * (a.ndim - 1))


def _rpad(n):
    return -(-n // R) * R


def _sc_gather(table, idx):
    V, D = table.shape
    B_in = idx.shape[0]
    info = plsc.get_sparse_core_info()
    NC, NS = info.num_cores, info.num_subcores
    NW = NC * NS
    CH = 128
    nch = -(-B_in // (NW * CH))
    B = NW * nch * CH
    idxp = jnp.pad(idx.astype(jnp.int32), (0, B - B_in))
    mesh = plsc.VectorSubcoreMesh(core_axis_name="c", subcore_axis_name="s")

    @functools.partial(
        pl.kernel, mesh=mesh,
        compiler_params=pltpu.CompilerParams(use_tc_tiling_on_sc=False),
        out_type=jax.ShapeDtypeStruct((B, D), jnp.float32),
        scratch_types=[pltpu.VMEM((CH,), jnp.int32),
                       pltpu.VMEM((CH, D), jnp.float32),
                       pltpu.SemaphoreType.DMA],
    )
    def gk(table_hbm, idx_hbm, out_hbm, idx_v, rows_v, sem):
        wid = lax.axis_index("s") * NC + lax.axis_index("c")

        def chunk(t, carry):
            base = (wid * nch + t) * CH
            pltpu.sync_copy(idx_hbm.at[pl.ds(base, CH)], idx_v)
            pltpu.async_copy(table_hbm.at[idx_v], rows_v, sem).wait()
            pltpu.sync_copy(rows_v, out_hbm.at[pl.ds(base, CH)])
            return carry

        lax.fori_loop(0, nch, chunk, 0)

    return gk(table, idxp)[:B_in]


def _knn(py_pad, pxT, exclude_self):
    ny_pad = py_pad.shape[0]
    nx = pxT.shape[1]
    grid = ny_pad // R

    def body(py_ref, pxT_ref, out_ref):
        i = pl.program_id(0)
        pyb = py_ref[...]
        pxt = pxT_ref[...]
        sy = jnp.sum(pyb * pyb, axis=1, keepdims=True)
        sx = jnp.sum(pxt * pxt, axis=0, keepdims=True)
        d = sy + sx - 2.0 * jnp.dot(pyb, pxt, preferred_element_type=jnp.float32)
        colio = lax.broadcasted_iota(jnp.int32, (R, nx), 1)
        if exclude_self:
            rows = i * R + lax.broadcasted_iota(jnp.int32, (R, 1), 0)
            d = jnp.where(colio == rows, jnp.inf, d)
        cols = []
        for _ in range(K):
            m = jnp.min(d, axis=1, keepdims=True)
            sel = jnp.min(jnp.where(d <= m, colio, _BIG), axis=1, keepdims=True)
            cols.append(sel)
            d = jnp.where(colio == sel, jnp.inf, d)
        out_ref[...] = jnp.concatenate(cols, axis=1)

    return pl.pallas_call(
        body,
        grid=(grid,),
        in_specs=[pl.BlockSpec((R, 3), lambda i: (i, 0)),
                  pl.BlockSpec((3, nx), lambda i: (0, 0))],
        out_specs=pl.BlockSpec((R, K), lambda i: (i, 0)),
        out_shape=jax.ShapeDtypeStruct((ny_pad, K), jnp.int32),
    )(py_pad, pxT)


def _fps(posT, m):
    n = posT.shape[1]

    def body(pt_ref, out_ref):
        pt = pt_ref[...]
        io = lax.broadcasted_iota(jnp.int32, (1, n), 1)
        d0 = jnp.sum((pt - pt[:, 0:1]) ** 2, axis=0, keepdims=True)
        out_ref[pl.ds(0, 1), :] = jnp.zeros((1, 1), jnp.int32)

        def step(i, dmin):
            dmax = jnp.max(dmin, axis=1, keepdims=True)
            sel = jnp.min(jnp.where(dmin >= dmax, io, _BIG), axis=1, keepdims=True)
            out_ref[pl.ds(i, 1), :] = sel
            pxn = jnp.sum(jnp.where(io == sel, pt, 0.0), axis=1, keepdims=True)
            dn = jnp.sum((pt - pxn) ** 2, axis=0, keepdims=True)
            return jnp.minimum(dmin, dn)

        lax.fori_loop(1, m, step, d0)

    return pl.pallas_call(
        body,
        out_shape=jax.ShapeDtypeStruct((m, 1), jnp.int32),
    )(posT)


def _dense_relu(x_pad, W, b):
    n_pad, cin = x_pad.shape
    cout = W.shape[1]
    b2 = b.reshape(1, cout)

    def body(x_ref, w_ref, b_ref, o_ref):
        o_ref[...] = jax.nn.relu(
            jnp.dot(x_ref[...], w_ref[...], preferred_element_type=jnp.float32)
            + b_ref[...])

    return pl.pallas_call(
        body,
        grid=(n_pad // R,),
        in_specs=[pl.BlockSpec((R, cin), lambda i: (i, 0)),
                  pl.BlockSpec((cin, cout), lambda i: (0, 0)),
                  pl.BlockSpec((1, cout), lambda i: (0, 0))],
        out_specs=pl.BlockSpec((R, cout), lambda i: (i, 0)),
        out_shape=jax.ShapeDtypeStruct((n_pad, cout), jnp.float32),
    )(x_pad, W, b2)


def _prep(x_pad, pos16, lin_in_W, lin_in_b, src_W, lin_W, dst_W):
    n_pad, c = x_pad.shape
    D = 2 * c + 16
    lb = lin_in_b.reshape(1, c)

    def body(x_ref, p_ref, liw_ref, lib_ref, sw_ref, lw_ref, dw_ref,
             tab_ref, xd_ref):
        h = jax.nn.relu(
            jnp.dot(x_ref[...], liw_ref[...], preferred_element_type=jnp.float32)
            + lib_ref[...])
        hs = jnp.dot(h, sw_ref[...], preferred_element_type=jnp.float32)
        hv = jnp.dot(h, lw_ref[...], preferred_element_type=jnp.float32)
        hd = jnp.dot(h, dw_ref[...], preferred_element_type=jnp.float32)
        tab_ref[...] = jnp.concatenate([hs, hv, p_ref[...]], axis=1)
        xd_ref[...] = hd

    return pl.pallas_call(
        body,
        grid=(n_pad // R,),
        in_specs=[pl.BlockSpec((R, c), lambda i: (i, 0)),
                  pl.BlockSpec((R, 16), lambda i: (i, 0)),
                  pl.BlockSpec((c, c), lambda i: (0, 0)),
                  pl.BlockSpec((1, c), lambda i: (0, 0)),
                  pl.BlockSpec((c, c), lambda i: (0, 0)),
                  pl.BlockSpec((c, c), lambda i: (0, 0)),
                  pl.BlockSpec((c, c), lambda i: (0, 0))],
        out_specs=[pl.BlockSpec((R, D), lambda i: (i, 0)),
                   pl.BlockSpec((R, c), lambda i: (i, 0))],
        out_shape=[jax.ShapeDtypeStruct((n_pad, D), jnp.float32),
                   jax.ShapeDtypeStruct((n_pad, c), jnp.float32)],
    )(x_pad, pos16, lin_in_W, lb, src_W, lin_W, dst_W)


def _conv(g, xd_pad, pos16, pW1, pb1, pW2, pb2, aW1, ab1, aW2, ab2,
          lout_W, lout_b):
    n_pad, c = xd_pad.shape
    D = 2 * c + 16
    pb1r = pb1.reshape(1, 64)
    pb2r = pb2.reshape(1, c)
    ab1r = ab1.reshape(1, 64)
    ab2r = ab2.reshape(1, c)
    lbr = lout_b.reshape(1, c)

    def body(g_ref, xd_ref, p_ref, pw1, pb1_, pw2, pb2_, aw1, ab1_, aw2, ab2_,
             lw, lb_, o_ref):
        gb = g_ref[...].reshape(E * R, D)
        xs_g = gb[:, :c]
        xv_g = gb[:, c:2 * c]
        pos_g = gb[:, 2 * c:2 * c + 3]
        pos_d = jnp.broadcast_to(p_ref[...][None, :, :3], (E, R, 3)).reshape(E * R, 3)
        rel = pos_d - pos_g
        t = jax.nn.relu(
            jnp.dot(rel, pw1[...], preferred_element_type=jnp.float32) + pb1_[...])
        delta = jax.nn.relu(
            jnp.dot(t, pw2[...], preferred_element_type=jnp.float32) + pb2_[...])
        xd_b = jnp.broadcast_to(xd_ref[...][None], (E, R, c)).reshape(E * R, c)
        ain = xd_b - xs_g + delta
        a1 = jax.nn.relu(
            jnp.dot(ain, aw1[...], preferred_element_type=jnp.float32) + ab1_[...])
        alpha = jax.nn.relu(
            jnp.dot(a1, aw2[...], preferred_element_type=jnp.float32) + ab2_[...])
        a3 = alpha.reshape(E, R, c)
        amax = jnp.max(a3, axis=0, keepdims=True)
        e3 = jnp.exp(a3 - amax)
        den = jnp.sum(e3, axis=0, keepdims=True)
        w3 = e3 / (den + 1e-16)
        v3 = (xv_g + delta).reshape(E, R, c)
        o = jnp.sum(w3 * v3, axis=0)
        o_ref[...] = jax.nn.relu(
            jnp.dot(o, lw[...], preferred_element_type=jnp.float32) + lb_[...])

    return pl.pallas_call(
        body,
        grid=(n_pad // R,),
        in_specs=[pl.BlockSpec((E, R, D), lambda i: (0, i, 0)),
                  pl.BlockSpec((R, c), lambda i: (i, 0)),
                  pl.BlockSpec((R, 16), lambda i: (i, 0)),
                  pl.BlockSpec((3, 64), lambda i: (0, 0)),
                  pl.BlockSpec((1, 64), lambda i: (0, 0)),
                  pl.BlockSpec((64, c), lambda i: (0, 0)),
                  pl.BlockSpec((1, c), lambda i: (0, 0)),
                  pl.BlockSpec((c, 64), lambda i: (0, 0)),
                  pl.BlockSpec((1, 64), lambda i: (0, 0)),
                  pl.BlockSpec((64, c), lambda i: (0, 0)),
                  pl.BlockSpec((1, c), lambda i: (0, 0)),
                  pl.BlockSpec((c, c), lambda i: (0, 0)),
                  pl.BlockSpec((1, c), lambda i: (0, 0))],
        out_specs=pl.BlockSpec((R, c), lambda i: (i, 0)),
        out_shape=jax.ShapeDtypeStruct((n_pad, c), jnp.float32),
    )(g, xd_pad, pos16, pW1, pb1r, pW2, pb2r, aW1, ab1r, aW2, ab2r,
      lout_W, lbr)


def _segmax(hg, m_pad, c):
    def body(g_ref, o_ref):
        o_ref[...] = jnp.max(g_ref[...], axis=0)

    return pl.pallas_call(
        body,
        grid=(m_pad // R,),
        in_specs=[pl.BlockSpec((K, R, c), lambda i: (0, i, 0))],
        out_specs=pl.BlockSpec((R, c), lambda i: (i, 0)),
        out_shape=jax.ShapeDtypeStruct((m_pad, c), jnp.float32),
    )(hg)


def _head(x, W1, b1, W2, b2):
    n, c = x.shape

    def body(x_ref, w1, b1_, w2, b2_, o_ref):
        mean = jnp.sum(x_ref[...], axis=0, keepdims=True) * (1.0 / n)
        h = jax.nn.relu(
            jnp.dot(mean, w1[...], preferred_element_type=jnp.float32) + b1_[...])
        o_ref[...] = jnp.dot(h, w2[...], preferred_element_type=jnp.float32) + b2_[...]

    return pl.pallas_call(
        body,
        out_shape=jax.ShapeDtypeStruct((1, W2.shape[1]), jnp.float32),
    )(x, W1, b1.reshape(1, -1), W2, b2.reshape(1, -1))


def _edge_major_idx(idx, n, n_pad):
    idxT = jnp.pad(idx.T, ((0, 0), (0, n_pad - n)))
    ar = jnp.arange(n_pad, dtype=jnp.int32)
    em = jnp.concatenate([idxT, ar[None]], axis=0)
    return jnp.where(ar[None, :] < n, em, 0)


def _tblock(x_pad, pos16, posT, n, n_pad, p, pref):
    idx = _knn(pos16[:, :3], posT, True)[:n]
    em = _edge_major_idx(idx, n, n_pad)
    table, xd = _prep(x_pad, pos16,
                      p[pref + "_lin_in_W"], p[pref + "_lin_in_b"],
                      p[pref + "_src_W"], p[pref + "_lin_W"], p[pref + "_dst_W"])
    D = table.shape[1]
    g = _sc_gather(table, em.reshape(-1)).reshape(E, n_pad, D)
    return _conv(g, xd, pos16,
                 p[pref + "_pos_W1"], p[pref + "_pos_b1"],
                 p[pref + "_pos_W2"], p[pref + "_pos_b2"],
                 p[pref + "_attn_W1"], p[pref + "_attn_b1"],
                 p[pref + "_attn_W2"], p[pref + "_attn_b2"],
                 p[pref + "_lin_out_W"], p[pref + "_lin_out_b"])


def kernel(x, pos, mi_W, mi_b, tb0_lin_in_W, tb0_lin_in_b, tb0_lin_out_W, tb0_lin_out_b, tb0_pos_W1, tb0_pos_b1, tb0_pos_W2, tb0_pos_b2, tb0_attn_W1, tb0_attn_b1, tb0_attn_W2, tb0_attn_b2, tb0_lin_W, tb0_src_W, tb0_dst_W, tb1_lin_in_W, tb1_lin_in_b, tb1_lin_out_W, tb1_lin_out_b, tb1_pos_W1, tb1_pos_b1, tb1_pos_W2, tb1_pos_b2, tb1_attn_W1, tb1_attn_b1, tb1_attn_W2, tb1_attn_b2, tb1_lin_W, tb1_src_W, tb1_dst_W, tb2_lin_in_W, tb2_lin_in_b, tb2_lin_out_W, tb2_lin_out_b, tb2_pos_W1, tb2_pos_b1, tb2_pos_W2, tb2_pos_b2, tb2_attn_W1, tb2_attn_b1, tb2_attn_W2, tb2_attn_b2, tb2_lin_W, tb2_src_W, tb2_dst_W, td0_W, td0_b, td1_W, td1_b, mo_W1, mo_b1, mo_W2, mo_b2):
    p = dict(locals())
    n0 = pos.shape[0]
    n0p = _rpad(n0)
    pos16_0 = jnp.pad(pos, ((0, n0p - n0), (0, 13)))
    posT0 = pos.T

    x0 = _dense_relu(_pad_rows(x, n0p), mi_W, mi_b)
    x1 = _tblock(x0, pos16_0, posT0, n0, n0p, p, "tb0")

    xcur, pos16, posT, n, n_pad = x1, pos16_0, posT0, n0, n0p
    for td, tb in (("td0", "tb1"), ("td1", "tb2")):
        m = int(n * 0.25)
        m_pad = _rpad(m)
        idc = _fps(posT, m)[:, 0]
        pos16_s = _sc_gather(pos16, idc)[:m]
        pos16_s = jnp.pad(pos16_s, ((0, m_pad - m), (0, 0)))
        posT_s = pos16_s[:m, :3].T
        col = _knn(pos16_s[:, :3], posT, False)[:m]
        colT = jnp.pad(col.T, ((0, 0), (0, m_pad - m)))
        colT = jnp.where(jnp.arange(m_pad)[None, :] < m, colT, 0)
        h = _dense_relu(xcur, p[td + "_W"], p[td + "_b"])
        c_out = h.shape[1]
        hg = _sc_gather(h, colT.reshape(-1)).reshape(K, m_pad, c_out)
        xds = _segmax(hg, m_pad, c_out)
        xcur = _tblock(xds, pos16_s, posT_s, m, m_pad, p, tb)
        pos16, posT, n, n_pad = pos16_s, posT_s, m, m_pad

    return _head(xcur[:n], mo_W1, mo_b1, mo_W2, mo_b2)


# speedup vs baseline: 10.3897x; 1.3318x over previous
import functools

import jax
import jax.numpy as jnp
from jax import lax
from jax.experimental import pallas as pl
from jax.experimental.pallas import tpu as pltpu
from jax.experimental.pallas import tpu_sc as plsc

R = 128
K = 16
E = K + 1
_BIG = 2 ** 30


def _pad_rows(a, n_pad):
    n = a.shape[0]
    if n == n_pad:
        return a
    return jnp.pad(a, ((0, n_pad - n),) + ((0, 0),) * (a.ndim - 1))


def _rpad(n):
    return -(-n // R) * R


def _sc_gather(table, idx):
    V, D = table.shape
    B_in = idx.shape[0]
    info = plsc.get_sparse_core_info()
    NC, NS = info.num_cores, info.num_subcores
    NW = NC * NS
    CH = 128
    nch = -(-B_in // (NW * CH))
    B = NW * nch * CH
    idxp = jnp.pad(idx.astype(jnp.int32), (0, B - B_in))
    mesh = plsc.VectorSubcoreMesh(core_axis_name="c", subcore_axis_name="s")

    @functools.partial(
        pl.kernel, mesh=mesh,
        compiler_params=pltpu.CompilerParams(use_tc_tiling_on_sc=False),
        out_type=jax.ShapeDtypeStruct((B, D), jnp.float32),
        scratch_types=[pltpu.VMEM((CH,), jnp.int32),
                       pltpu.VMEM((CH, D), jnp.float32),
                       pltpu.SemaphoreType.DMA],
    )
    def gk(table_hbm, idx_hbm, out_hbm, idx_v, rows_v, sem):
        wid = lax.axis_index("s") * NC + lax.axis_index("c")

        def chunk(t, carry):
            base = (wid * nch + t) * CH
            pltpu.sync_copy(idx_hbm.at[pl.ds(base, CH)], idx_v)
            pltpu.async_copy(table_hbm.at[idx_v], rows_v, sem).wait()
            pltpu.sync_copy(rows_v, out_hbm.at[pl.ds(base, CH)])
            return carry

        lax.fori_loop(0, nch, chunk, 0)

    return gk(table, idxp)[:B_in]


def _knn(py_pad, pxT, exclude_self):
    ny_pad = py_pad.shape[0]
    nx = pxT.shape[1]
    grid = ny_pad // R

    def body(py_ref, pxT_ref, out_ref):
        i = pl.program_id(0)
        pyb = py_ref[...]
        pxt = pxT_ref[...]
        sy = jnp.sum(pyb * pyb, axis=1, keepdims=True)
        sx = jnp.sum(pxt * pxt, axis=0, keepdims=True)
        d = sy + sx - 2.0 * jnp.dot(pyb, pxt, preferred_element_type=jnp.float32)
        colio = lax.broadcasted_iota(jnp.int32, (R, nx), 1)
        if exclude_self:
            rows = i * R + lax.broadcasted_iota(jnp.int32, (R, 1), 0)
            d = jnp.where(colio == rows, jnp.inf, d)
        cols = []
        for k in range(K):
            m = jnp.min(d, axis=1, keepdims=True)
            hit = d <= m
            sel = jnp.min(jnp.where(hit, colio, _BIG), axis=1, keepdims=True)
            cols.append(sel)
            if k < K - 1:
                d = jnp.where(hit, jnp.inf, d)
        out_ref[...] = jnp.concatenate(cols, axis=1)

    return pl.pallas_call(
        body,
        grid=(grid,),
        in_specs=[pl.BlockSpec((R, 3), lambda i: (i, 0)),
                  pl.BlockSpec((3, nx), lambda i: (0, 0))],
        out_specs=pl.BlockSpec((R, K), lambda i: (i, 0)),
        out_shape=jax.ShapeDtypeStruct((ny_pad, K), jnp.int32),
    )(py_pad, pxT)


def _fps(pos24, n, m):
    G = pos24.shape[1]

    def body(p_ref, out_ref):
        p24 = p_ref[...]
        px, py, pz = p24[0:8], p24[8:16], p24[16:24]
        io = (lax.broadcasted_iota(jnp.int32, (8, G), 0) * G
              + lax.broadcasted_iota(jnp.int32, (8, G), 1))
        valid = io < n

        def q_of(oh):
            qx = jnp.sum(jnp.sum(jnp.where(oh, px, 0.0), axis=1, keepdims=True),
                         axis=0, keepdims=True)
            qy = jnp.sum(jnp.sum(jnp.where(oh, py, 0.0), axis=1, keepdims=True),
                         axis=0, keepdims=True)
            qz = jnp.sum(jnp.sum(jnp.where(oh, pz, 0.0), axis=1, keepdims=True),
                         axis=0, keepdims=True)
            return qx, qy, qz

        def d_to(qx, qy, qz):
            return ((px - qx) ** 2 + (py - qy) ** 2 + (pz - qz) ** 2)

        qx, qy, qz = q_of(io == 0)
        d0 = jnp.where(valid, d_to(qx, qy, qz), -jnp.inf)
        out_ref[pl.ds(0, 1), :] = jnp.zeros((1, 1), jnp.int32)

        def step(i, dmin):
            dmax = jnp.max(jnp.max(dmin, axis=1, keepdims=True),
                           axis=0, keepdims=True)
            sel = jnp.min(jnp.min(jnp.where(dmin >= dmax, io, _BIG),
                                  axis=1, keepdims=True), axis=0, keepdims=True)
            out_ref[pl.ds(i, 1), :] = sel
            qx, qy, qz = q_of(io == sel)
            return jnp.minimum(dmin, d_to(qx, qy, qz))

        lax.fori_loop(1, m, step, d0)

    return pl.pallas_call(
        body,
        out_shape=jax.ShapeDtypeStruct((m, 1), jnp.int32),
    )(pos24)


def _dense_relu(x_pad, W, b):
    n_pad, cin = x_pad.shape
    cout = W.shape[1]
    b2 = b.reshape(1, cout)

    def body(x_ref, w_ref, b_ref, o_ref):
        o_ref[...] = jax.nn.relu(
            jnp.dot(x_ref[...], w_ref[...], preferred_element_type=jnp.float32)
            + b_ref[...])

    return pl.pallas_call(
        body,
        grid=(n_pad // R,),
        in_specs=[pl.BlockSpec((R, cin), lambda i: (i, 0)),
                  pl.BlockSpec((cin, cout), lambda i: (0, 0)),
                  pl.BlockSpec((1, cout), lambda i: (0, 0))],
        out_specs=pl.BlockSpec((R, cout), lambda i: (i, 0)),
        out_shape=jax.ShapeDtypeStruct((n_pad, cout), jnp.float32),
    )(x_pad, W, b2)


def _prep(x_pad, pos16, lin_in_W, lin_in_b, src_W, lin_W, dst_W):
    n_pad, c = x_pad.shape
    D = 2 * c + 16
    lb = lin_in_b.reshape(1, c)

    def body(x_ref, p_ref, liw_ref, lib_ref, sw_ref, lw_ref, dw_ref,
             tab_ref, xd_ref):
        h = jax.nn.relu(
            jnp.dot(x_ref[...], liw_ref[...], preferred_element_type=jnp.float32)
            + lib_ref[...])
        hs = jnp.dot(h, sw_ref[...], preferred_element_type=jnp.float32)
        hv = jnp.dot(h, lw_ref[...], preferred_element_type=jnp.float32)
        hd = jnp.dot(h, dw_ref[...], preferred_element_type=jnp.float32)
        tab_ref[...] = jnp.concatenate([hs, hv, p_ref[...]], axis=1)
        xd_ref[...] = hd

    return pl.pallas_call(
        body,
        grid=(n_pad // R,),
        in_specs=[pl.BlockSpec((R, c), lambda i: (i, 0)),
                  pl.BlockSpec((R, 16), lambda i: (i, 0)),
                  pl.BlockSpec((c, c), lambda i: (0, 0)),
                  pl.BlockSpec((1, c), lambda i: (0, 0)),
                  pl.BlockSpec((c, c), lambda i: (0, 0)),
                  pl.BlockSpec((c, c), lambda i: (0, 0)),
                  pl.BlockSpec((c, c), lambda i: (0, 0))],
        out_specs=[pl.BlockSpec((R, D), lambda i: (i, 0)),
                   pl.BlockSpec((R, c), lambda i: (i, 0))],
        out_shape=[jax.ShapeDtypeStruct((n_pad, D), jnp.float32),
                   jax.ShapeDtypeStruct((n_pad, c), jnp.float32)],
    )(x_pad, pos16, lin_in_W, lb, src_W, lin_W, dst_W)


def _conv(g, xd_pad, pos16, pW1, pb1, pW2, pb2, aW1, ab1, aW2, ab2,
          lout_W, lout_b):
    n_pad, c = xd_pad.shape
    D = 2 * c + 16
    pb1r = pb1.reshape(1, 64)
    pb2r = pb2.reshape(1, c)
    ab1r = ab1.reshape(1, 64)
    ab2r = ab2.reshape(1, c)
    lbr = lout_b.reshape(1, c)

    def body(g_ref, xd_ref, p_ref, pw1, pb1_, pw2, pb2_, aw1, ab1_, aw2, ab2_,
             lw, lb_, o_ref):
        gb = g_ref[...].reshape(E * R, D)
        xs_g = gb[:, :c]
        xv_g = gb[:, c:2 * c]
        pos_g = gb[:, 2 * c:2 * c + 3]
        pos_d = jnp.broadcast_to(p_ref[...][None, :, :3], (E, R, 3)).reshape(E * R, 3)
        rel = pos_d - pos_g
        t = jax.nn.relu(
            jnp.dot(rel, pw1[...], preferred_element_type=jnp.float32) + pb1_[...])
        delta = jax.nn.relu(
            jnp.dot(t, pw2[...], preferred_element_type=jnp.float32) + pb2_[...])
        xd_b = jnp.broadcast_to(xd_ref[...][None], (E, R, c)).reshape(E * R, c)
        ain = xd_b - xs_g + delta
        a1 = jax.nn.relu(
            jnp.dot(ain, aw1[...], preferred_element_type=jnp.float32) + ab1_[...])
        alpha = jax.nn.relu(
            jnp.dot(a1, aw2[...], preferred_element_type=jnp.float32) + ab2_[...])
        a3 = alpha.reshape(E, R, c)
        amax = jnp.max(a3, axis=0, keepdims=True)
        e3 = jnp.exp(a3 - amax)
        den = jnp.sum(e3, axis=0, keepdims=True)
        w3 = e3 / (den + 1e-16)
        v3 = (xv_g + delta).reshape(E, R, c)
        o = jnp.sum(w3 * v3, axis=0)
        o_ref[...] = jax.nn.relu(
            jnp.dot(o, lw[...], preferred_element_type=jnp.float32) + lb_[...])

    return pl.pallas_call(
        body,
        grid=(n_pad // R,),
        in_specs=[pl.BlockSpec((E, R, D), lambda i: (0, i, 0)),
                  pl.BlockSpec((R, c), lambda i: (i, 0)),
                  pl.BlockSpec((R, 16), lambda i: (i, 0)),
                  pl.BlockSpec((3, 64), lambda i: (0, 0)),
                  pl.BlockSpec((1, 64), lambda i: (0, 0)),
                  pl.BlockSpec((64, c), lambda i: (0, 0)),
                  pl.BlockSpec((1, c), lambda i: (0, 0)),
                  pl.BlockSpec((c, 64), lambda i: (0, 0)),
                  pl.BlockSpec((1, 64), lambda i: (0, 0)),
                  pl.BlockSpec((64, c), lambda i: (0, 0)),
                  pl.BlockSpec((1, c), lambda i: (0, 0)),
                  pl.BlockSpec((c, c), lambda i: (0, 0)),
                  pl.BlockSpec((1, c), lambda i: (0, 0))],
        out_specs=pl.BlockSpec((R, c), lambda i: (i, 0)),
        out_shape=jax.ShapeDtypeStruct((n_pad, c), jnp.float32),
    )(g, xd_pad, pos16, pW1, pb1r, pW2, pb2r, aW1, ab1r, aW2, ab2r,
      lout_W, lbr)


def _segmax(hg, m_pad, c):
    def body(g_ref, o_ref):
        o_ref[...] = jnp.max(g_ref[...], axis=0)

    return pl.pallas_call(
        body,
        grid=(m_pad // R,),
        in_specs=[pl.BlockSpec((K, R, c), lambda i: (0, i, 0))],
        out_specs=pl.BlockSpec((R, c), lambda i: (i, 0)),
        out_shape=jax.ShapeDtypeStruct((m_pad, c), jnp.float32),
    )(hg)


def _head(x, W1, b1, W2, b2):
    n, c = x.shape

    def body(x_ref, w1, b1_, w2, b2_, o_ref):
        mean = jnp.sum(x_ref[...], axis=0, keepdims=True) * (1.0 / n)
        h = jax.nn.relu(
            jnp.dot(mean, w1[...], preferred_element_type=jnp.float32) + b1_[...])
        o_ref[...] = jnp.dot(h, w2[...], preferred_element_type=jnp.float32) + b2_[...]

    return pl.pallas_call(
        body,
        out_shape=jax.ShapeDtypeStruct((1, W2.shape[1]), jnp.float32),
    )(x, W1, b1.reshape(1, -1), W2, b2.reshape(1, -1))


def _edge_major_idx(idx, n, n_pad):
    idxT = jnp.pad(idx.T, ((0, 0), (0, n_pad - n)))
    ar = jnp.arange(n_pad, dtype=jnp.int32)
    em = jnp.concatenate([idxT, ar[None]], axis=0)
    return jnp.where(ar[None, :] < n, em, 0)


def _pack24(pos16, n):
    G = -(-n // 8)
    comps = [jnp.pad(pos16[:n, i], (0, 8 * G - n)).reshape(8, G)
             for i in range(3)]
    return jnp.concatenate(comps, axis=0)


def _tblock(x_pad, pos16, idx, n, n_pad, p, pref):
    em = _edge_major_idx(idx, n, n_pad)
    table, xd = _prep(x_pad, pos16,
                      p[pref + "_lin_in_W"], p[pref + "_lin_in_b"],
                      p[pref + "_src_W"], p[pref + "_lin_W"], p[pref + "_dst_W"])
    D = table.shape[1]
    g = _sc_gather(table, em.reshape(-1)).reshape(E, n_pad, D)
    return _conv(g, xd, pos16,
                 p[pref + "_pos_W1"], p[pref + "_pos_b1"],
                 p[pref + "_pos_W2"], p[pref + "_pos_b2"],
                 p[pref + "_attn_W1"], p[pref + "_attn_b1"],
                 p[pref + "_attn_W2"], p[pref + "_attn_b2"],
                 p[pref + "_lin_out_W"], p[pref + "_lin_out_b"])


def kernel(x, pos, mi_W, mi_b, tb0_lin_in_W, tb0_lin_in_b, tb0_lin_out_W, tb0_lin_out_b, tb0_pos_W1, tb0_pos_b1, tb0_pos_W2, tb0_pos_b2, tb0_attn_W1, tb0_attn_b1, tb0_attn_W2, tb0_attn_b2, tb0_lin_W, tb0_src_W, tb0_dst_W, tb1_lin_in_W, tb1_lin_in_b, tb1_lin_out_W, tb1_lin_out_b, tb1_pos_W1, tb1_pos_b1, tb1_pos_W2, tb1_pos_b2, tb1_attn_W1, tb1_attn_b1, tb1_attn_W2, tb1_attn_b2, tb1_lin_W, tb1_src_W, tb1_dst_W, tb2_lin_in_W, tb2_lin_in_b, tb2_lin_out_W, tb2_lin_out_b, tb2_pos_W1, tb2_pos_b1, tb2_pos_W2, tb2_pos_b2, tb2_attn_W1, tb2_attn_b1, tb2_attn_W2, tb2_attn_b2, tb2_lin_W, tb2_src_W, tb2_dst_W, td0_W, td0_b, td1_W, td1_b, mo_W1, mo_b1, mo_W2, mo_b2):
    p = dict(locals())
    n0 = pos.shape[0]
    n0p = _rpad(n0)
    pos16_0 = jnp.pad(pos, ((0, n0p - n0), (0, 13)))
    posT0 = pos.T

    x0 = _dense_relu(_pad_rows(x, n0p), mi_W, mi_b)
    idx0 = _knn(pos16_0[:, :3], posT0, True)[:n0]
    x1 = _tblock(x0, pos16_0, idx0, n0, n0p, p, "tb0")

    xcur, pos16, idx_cur, n, n_pad = x1, pos16_0, idx0, n0, n0p
    for td, tb in (("td0", "tb1"), ("td1", "tb2")):
        m = int(n * 0.25)
        m_pad = _rpad(m)
        idc = _fps(_pack24(pos16, n), n, m)[:, 0]
        pos16_s = _sc_gather(pos16, idc)[:m]
        pos16_s = jnp.pad(pos16_s, ((0, m_pad - m), (0, 0)))
        idxf = lax.bitcast_convert_type(idx_cur, jnp.float32)
        nbr = lax.bitcast_convert_type(_sc_gather(idxf, idc)[:m], jnp.int32)
        col = jnp.concatenate([idc[:, None], nbr[:, :15]], axis=1)
        colT = jnp.pad(col.T, ((0, 0), (0, m_pad - m)))
        colT = jnp.where(jnp.arange(m_pad)[None, :] < m, colT, 0)
        h = _dense_relu(xcur, p[td + "_W"], p[td + "_b"])
        c_out = h.shape[1]
        hg = _sc_gather(h, colT.reshape(-1)).reshape(K, m_pad, c_out)
        xds = _segmax(hg, m_pad, c_out)
        idx_s = _knn(pos16_s[:, :3], pos16_s[:m, :3].T, True)[:m]
        xcur = _tblock(xds, pos16_s, idx_s, m, m_pad, p, tb)
        pos16, idx_cur, n, n_pad = pos16_s, idx_s, m, m_pad

    return _head(xcur[:n], mo_W1, mo_b1, mo_W2, mo_b2)
